# Initial kernel scaffold; baseline (speedup 1.0000x reference)
import jax
import jax.numpy as jnp
from jax.experimental import pallas as pl


def kernel(x, edge_index, W0, al0, ar0, bn0_g, bn0_b, W1, al1, ar1, bn1_g, bn1_b, W2, al2, ar2, bias_last):
    raise NotImplementedError("write your pallas kernel here")


# speedup vs baseline: 1.1227x; 1.1227x over previous
import functools

import jax
import jax.numpy as jnp
from jax.experimental import pallas as pl
from jax.experimental.pallas import tpu as pltpu

N = 10000
E = 320000
D = 128


def _dense_stage_kernel(h_ref, W_ref, al_ref, ar_ref, h_out_ref, el_ref, er_ref):
    h = h_ref[...]
    W = W_ref[...]
    hw = jnp.dot(h, W, preferred_element_type=jnp.float32)
    h_out_ref[...] = hw
    el_ref[...] = hw @ al_ref[...].reshape(D, 1)
    er_ref[...] = hw @ ar_ref[...].reshape(D, 1)


def _dense_stage(h, W, al, ar):
    return pl.pallas_call(
        _dense_stage_kernel,
        out_shape=(
            jax.ShapeDtypeStruct((N, D), jnp.float32),
            jax.ShapeDtypeStruct((N, 1), jnp.float32),
            jax.ShapeDtypeStruct((N, 1), jnp.float32),
        ),
    )(h, W, al, ar)


def _bn_relu_kernel(x_ref, g_ref, b_ref, o_ref):
    x = x_ref[...]
    mu = jnp.mean(x, axis=0, keepdims=True)
    var = jnp.mean((x - mu) ** 2, axis=0, keepdims=True)
    y = (x - mu) / jnp.sqrt(var + 1e-5) * g_ref[...] + b_ref[...]
    o_ref[...] = jnp.maximum(y, 0.0)


def _bn_relu(x, g, b):
    return pl.pallas_call(
        _bn_relu_kernel,
        out_shape=jax.ShapeDtypeStruct((N, D), jnp.float32),
    )(x, g.reshape(1, D), b.reshape(1, D))


def _gat_edges(h, el, er, src, dst):
    e = jax.nn.leaky_relu(el[src] + er[dst], negative_slope=0.2)
    m = jax.ops.segment_max(e, dst, num_segments=N)
    m = jnp.where(jnp.isfinite(m), m, 0.0)
    ex = jnp.exp(e - m[dst])
    s = jax.ops.segment_sum(ex, dst, num_segments=N)
    alpha = ex / (s[dst] + 1e-9)
    return jax.ops.segment_sum(h[src] * alpha[:, None], dst, num_segments=N)


def kernel(x, edge_index, W0, al0, ar0, bn0_g, bn0_b, W1, al1, ar1, bn1_g,
           bn1_b, W2, al2, ar2, bias_last):
    src = edge_index[0]
    dst = edge_index[1]

    h, el, er = _dense_stage(x, W0, al0, ar0)
    h = _gat_edges(h, el[:, 0], er[:, 0], src, dst)
    h = _bn_relu(h, bn0_g, bn0_b)

    h, el, er = _dense_stage(h, W1, al1, ar1)
    h = _gat_edges(h, el[:, 0], er[:, 0], src, dst)
    h = _bn_relu(h, bn1_g, bn1_b)

    h, el, er = _dense_stage(h, W2, al2, ar2)
    h = _gat_edges(h, el[:, 0], er[:, 0], src, dst)
    return h + bias_last


# speedup vs baseline: 29.8373x; 26.5759x over previous
import functools

import jax
import jax.numpy as jnp
from jax import lax
from jax.experimental import pallas as pl
from jax.experimental.pallas import tpu as pltpu
from jax.experimental.pallas import tpu_sc as plsc

N = 10000
E = 320000
D = 128

NC = 2
NS = 16
ET = E // NS
K = 80
CH = ET // K
SEG = 5
CHS = CH // SEG
NPAD = 10240
NH = NPAD // NC
SLC = NPAD // NS
RPT = NH // NS
EXP = 20480


def _b16(i):
    return lax.broadcast_in_dim(jnp.int32(i) if isinstance(i, int) else i,
                                (16,), ())


def _sc_edge_kernel(h_hbm, el_hbm, er_hbm, src_hbm, dst_hbm, c_hbm,
                    out_hbm, s_hbm,
                    src_v, dst_v, el_v, er_v, s_loc, rows_v, c_v,
                    ex_c, dloc_v, acc_sh, sem):
    c = lax.axis_index("c")
    s = lax.axis_index("s")
    base_node = c * NH

    pltpu.sync_copy(el_hbm, el_v)
    pltpu.sync_copy(er_hbm, er_v)
    pltpu.sync_copy(c_hbm.at[pl.ds(0, 16)], c_v)

    zero16 = jnp.zeros((16,), jnp.float32)

    def _zs(j, _):
        s_loc[pl.ds(j * 16, 16)] = zero16
        return ()
    lax.fori_loop(0, NPAD // 16, _zs, (), unroll=8)

    def _zr(r, _):
        for cc in range(8):
            rows_v[0, r, pl.ds(cc * 16, 16)] = zero16
        return ()
    lax.fori_loop(0, K, _zr, (), unroll=4)

    for rc in range(RPT // K):
        pltpu.sync_copy(rows_v.at[0], acc_sh.at[pl.ds(s * RPT + rc * K, K)])
    plsc.subcore_barrier()

    cvec = c_v[...]
    b16 = _b16(base_node)

    def _compute_w(j, p):
        for i in range(K // 16):
            isrc = src_v[j, pl.ds(i * 16, 16)]
            idst = dst_v[j, pl.ds(i * 16, 16)]
            ev = plsc.load_gather(el_v, [isrc]) + plsc.load_gather(er_v, [idst])
            ev = jnp.where(ev >= 0.0, ev, 0.2 * ev)
            exv = jnp.exp(ev - cvec)
            loc = idst - b16
            mine = (loc >= 0) & (loc < NH)
            exm = jnp.where(mine, exv, 0.0)
            plsc.addupdate_scatter(s_loc, [idst], exm)
            ex_c[p, pl.ds(i * 16, 16)] = exm
            dloc_v[p, pl.ds(i * 16, 16)] = jnp.where(mine, loc, 0)

    def _drain(p):
        for i in range(K // 16):
            w16 = ex_c[p, pl.ds(i * 16, 16)]
            for l in range(16):
                r = i * 16 + l
                wv = lax.broadcast_in_dim(w16[l], (16,), ())
                for cc in range(8):
                    rows_v[p, r, pl.ds(cc * 16, 16)] = (
                        rows_v[p, r, pl.ds(cc * 16, 16)] * wv)
        pltpu.sync_copy(rows_v.at[p], acc_sh.at[dloc_v.at[p]], add=True)

    for seg in range(SEG):
        pltpu.sync_copy(src_hbm.at[s, seg], src_v)
        pltpu.sync_copy(dst_hbm.at[s, seg], dst_v)

        pltpu.async_copy(h_hbm.at[src_v.at[0]], rows_v.at[0], sem)
        _compute_w(0, 0)

        def _loop(j, _):
            p = lax.rem(j, 2)
            q = 1 - p
            pltpu.async_copy(h_hbm.at[src_v.at[j]], rows_v.at[p], sem)
            _compute_w(j, p)
            pltpu.make_async_copy(h_hbm.at[src_v.at[j - 1]],
                                  rows_v.at[q], sem).wait()
            _drain(q)
            return ()
        lax.fori_loop(1, CHS, _loop, ())

        lastp = (CHS - 1) % 2
        pltpu.make_async_copy(h_hbm.at[src_v.at[CHS - 1]],
                              rows_v.at[lastp], sem).wait()
        _drain(lastp)

    pltpu.sync_copy(s_loc, s_hbm.at[c, s])

    plsc.subcore_barrier()

    pltpu.sync_copy(acc_sh.at[pl.ds(s * RPT, RPT)],
                    out_hbm.at[pl.ds(c * NH + s * RPT, RPT)])


@functools.partial(
    pl.kernel,
    out_type=(
        jax.ShapeDtypeStruct((NPAD, D), jnp.float32),
        jax.ShapeDtypeStruct((NC, NS, NPAD), jnp.float32),
    ),
    mesh=plsc.VectorSubcoreMesh(core_axis_name="c", subcore_axis_name="s"),
    compiler_params=pltpu.CompilerParams(needs_layout_passes=False,
                                         use_tc_tiling_on_sc=False),
    scratch_types=[
        pltpu.VMEM((CHS, K), jnp.int32),
        pltpu.VMEM((CHS, K), jnp.int32),
        pltpu.VMEM((NPAD,), jnp.float32),
        pltpu.VMEM((NPAD,), jnp.float32),
        pltpu.VMEM((NPAD,), jnp.float32),
        pltpu.VMEM((2, K, D), jnp.float32),
        pltpu.VMEM((16,), jnp.float32),
        pltpu.VMEM((2, 128), jnp.float32),
        pltpu.VMEM((2, K), jnp.int32),
        pltpu.VMEM_SHARED((NH, D), jnp.float32),
        pltpu.SemaphoreType.DMA,
    ],
)
def _sc_edge(h_hbm, el_hbm, er_hbm, src_hbm, dst_hbm, c_hbm, out_hbm, s_hbm,
             *scratch):
    _sc_edge_kernel(h_hbm, el_hbm, er_hbm, src_hbm, dst_hbm, c_hbm,
                    out_hbm, s_hbm, *scratch)


def _dense0_kernel(x_ref, W_ref, al_ref, ar_ref, h_ref, el_ref, er_ref,
                   c_ref):
    h = jnp.dot(x_ref[...], W_ref[...], preferred_element_type=jnp.float32)
    h_ref[...] = h
    el = h @ al_ref[...].reshape(D, 1)
    er = h @ ar_ref[...].reshape(D, 1)
    el_ref[...] = el
    er_ref[...] = er
    cub = jnp.max(el) + jnp.max(er)
    cub = jnp.where(cub >= 0.0, cub, 0.2 * cub)
    c_ref[...] = jnp.full((1, 128), cub, jnp.float32)


def _dense0(x, W, al, ar):
    return pl.pallas_call(
        _dense0_kernel,
        out_shape=(
            jax.ShapeDtypeStruct((N, D), jnp.float32),
            jax.ShapeDtypeStruct((N, 1), jnp.float32),
            jax.ShapeDtypeStruct((N, 1), jnp.float32),
            jax.ShapeDtypeStruct((1, 128), jnp.float32),
        ),
    )(x, W, al, ar)


def _mid_kernel(a_ref, s_ref, g_ref, b_ref, W_ref, al_ref, ar_ref,
                h_ref, el_ref, er_ref, c_ref):
    stot = jnp.sum(s_ref[...], axis=0)
    recip = jnp.where(stot > 0.0, 1.0 / jnp.where(stot > 0.0, stot, 1.0), 0.0)
    agg = a_ref[...] * recip[:, None]
    mu = jnp.mean(agg, axis=0, keepdims=True)
    var = jnp.mean((agg - mu) ** 2, axis=0, keepdims=True)
    hin = (agg - mu) / jnp.sqrt(var + 1e-5) * g_ref[...] + b_ref[...]
    hin = jnp.maximum(hin, 0.0)
    h = jnp.dot(hin, W_ref[...], preferred_element_type=jnp.float32)
    h_ref[...] = h
    el = h @ al_ref[...].reshape(D, 1)
    er = h @ ar_ref[...].reshape(D, 1)
    el_ref[...] = el
    er_ref[...] = er
    cub = jnp.max(el) + jnp.max(er)
    cub = jnp.where(cub >= 0.0, cub, 0.2 * cub)
    c_ref[...] = jnp.full((1, 128), cub, jnp.float32)


def _mid(acc, s_part, g, b, W, al, ar):
    return pl.pallas_call(
        _mid_kernel,
        out_shape=(
            jax.ShapeDtypeStruct((N, D), jnp.float32),
            jax.ShapeDtypeStruct((N, 1), jnp.float32),
            jax.ShapeDtypeStruct((N, 1), jnp.float32),
            jax.ShapeDtypeStruct((1, 128), jnp.float32),
        ),
    )(acc[:N, :], s_part.reshape(NC * NS, NPAD)[:, :N], g.reshape(1, D),
      b.reshape(1, D), W, al, ar)


def _final_kernel(a_ref, s_ref, bias_ref, o_ref):
    stot = jnp.sum(s_ref[...], axis=0)
    recip = jnp.where(stot > 0.0, 1.0 / jnp.where(stot > 0.0, stot, 1.0), 0.0)
    o_ref[...] = a_ref[...] * recip[:, None] + bias_ref[...]


def _final(acc, s_part, bias):
    return pl.pallas_call(
        _final_kernel,
        out_shape=jax.ShapeDtypeStruct((N, D), jnp.float32),
    )(acc[:N, :], s_part.reshape(NC * NS, NPAD)[:, :N], bias.reshape(1, D))


def _padn(v):
    return jnp.pad(v.reshape(N), (0, NPAD - N))


def kernel(x, edge_index, W0, al0, ar0, bn0_g, bn0_b, W1, al1, ar1, bn1_g,
           bn1_b, W2, al2, ar2, bias_last):
    src = edge_index[0].reshape(NS, SEG, CHS, K)
    dst = edge_index[1].reshape(NS, SEG, CHS, K)

    h, el, er, c0 = _dense0(x, W0, al0, ar0)
    acc, s_part = _sc_edge(h, _padn(el), _padn(er), src, dst, c0.reshape(128))

    h, el, er, c1 = _mid(acc, s_part, bn0_g, bn0_b, W1, al1, ar1)
    acc, s_part = _sc_edge(h, _padn(el), _padn(er), src, dst, c1.reshape(128))

    h, el, er, c2 = _mid(acc, s_part, bn1_g, bn1_b, W2, al2, ar2)
    acc, s_part = _sc_edge(h, _padn(el), _padn(er), src, dst, c2.reshape(128))

    return _final(acc, s_part, bias_last)
